# Initial kernel scaffold; baseline (speedup 1.0000x reference)
import jax
import jax.numpy as jnp
from jax.experimental import pallas as pl


def kernel(x, edge_index, W_self, W_neigh):
    raise NotImplementedError("write your pallas kernel here")


# speedup vs baseline: 3.3964x; 3.3964x over previous
import functools

import jax
import jax.numpy as jnp
from jax import lax
from jax.experimental import pallas as pl
from jax.experimental.pallas import tpu as pltpu
from jax.experimental.pallas import tpu_sc as plsc

N_NODES = 10000
D = 128
NC, NS = 2, 16
NW = NC * NS
EW = 10240
C = 128
NCH = EW // C
ACC = 10240
RPT = ACC // NS
CW = 16
BLK = 1000

_MESH = plsc.VectorSubcoreMesh(core_axis_name="c", subcore_axis_name="s")
_SC_PARAMS = pltpu.CompilerParams(use_tc_tiling_on_sc=False)


def _sc_feature_sums(x_hbm, idx_hbm, sums_hbm,
                     idx_v, src_v, dst_v, buf_v, sem, acc_sh):
    cid = lax.axis_index("c")
    sid = lax.axis_index("s")
    wid = cid * NS + sid

    pltpu.sync_copy(idx_hbm.at[wid], idx_v)

    def _unpack(i, _):
        r = i // (C // 16)
        c = (i % (C // 16)) * 16
        v = idx_v[r, pl.ds(c, 16)]
        src_v[r, pl.ds(c, 16)] = jnp.bitwise_and(v, 0xFFFF)
        dst_v[r, pl.ds(c, 16)] = lax.shift_right_logical(v, 16)
        return 0
    lax.fori_loop(0, NCH * (C // 16), _unpack, 0)

    def _zrow(i, _):
        r = i // (D // 16)
        c = (i % (D // 16)) * 16
        buf_v[r, pl.ds(c, 16)] = jnp.zeros((16,), jnp.float32)
        return 0
    lax.fori_loop(0, C * (D // 16), _zrow, 0)

    base = sid * RPT
    for i in range(RPT // C):
        pltpu.sync_copy(buf_v, acc_sh.at[pl.ds(base + i * C, C)])
    plsc.subcore_barrier()

    def _chunk(j, _):
        pltpu.async_copy(x_hbm.at[src_v.at[j]], buf_v, sem).wait()
        pltpu.sync_copy(buf_v, acc_sh.at[dst_v.at[j]], add=True)
        return 0
    lax.fori_loop(0, NCH, _chunk, 0)

    plsc.subcore_barrier()

    pltpu.sync_copy(acc_sh.at[pl.ds(base, RPT)],
                    sums_hbm.at[cid, pl.ds(base, RPT)])


def _sc_counts(idx_hbm, cnts_hbm, idx_v, dst_v, ones_v, z16_v, cnt_sh):
    cid = lax.axis_index("c")
    sid = lax.axis_index("s")
    wid = cid * NS + sid

    pltpu.sync_copy(idx_hbm.at[wid], idx_v)

    def _unpack(i, _):
        r = i // (C // 16)
        c = (i % (C // 16)) * 16
        dst_v[r, pl.ds(c, 16)] = lax.shift_right_logical(
            idx_v[r, pl.ds(c, 16)], 16)
        return 0
    lax.fori_loop(0, NCH * (C // 16), _unpack, 0)

    def _fill(i, _):
        ones_v[i, :] = jnp.ones((CW,), jnp.float32)
        return 0
    lax.fori_loop(0, C, _fill, 0)

    def _z16(i, _):
        z16_v[i, :] = jnp.zeros((CW,), jnp.float32)
        return 0
    lax.fori_loop(0, RPT, _z16, 0)

    base = sid * RPT
    pltpu.sync_copy(z16_v, cnt_sh.at[pl.ds(base, RPT)])
    plsc.subcore_barrier()

    def _chunk(j, _):
        pltpu.sync_copy(ones_v, cnt_sh.at[dst_v.at[j]], add=True)
        return 0
    lax.fori_loop(0, NCH, _chunk, 0)

    plsc.subcore_barrier()

    pltpu.sync_copy(cnt_sh.at[pl.ds(base, RPT)],
                    cnts_hbm.at[cid, pl.ds(base, RPT)])


_sums_call = functools.partial(
    pl.kernel,
    mesh=_MESH,
    compiler_params=_SC_PARAMS,
    out_type=[jax.ShapeDtypeStruct((NC, ACC, D), jnp.float32)],
    scratch_types=[
        pltpu.VMEM((NCH, C), jnp.int32),
        pltpu.VMEM((NCH, C), jnp.int32),
        pltpu.VMEM((NCH, C), jnp.int32),
        pltpu.VMEM((C, D), jnp.float32),
        pltpu.SemaphoreType.DMA,
        pltpu.VMEM_SHARED((ACC, D), jnp.float32),
    ],
)(_sc_feature_sums)

_cnts_call = functools.partial(
    pl.kernel,
    mesh=_MESH,
    compiler_params=_SC_PARAMS,
    out_type=[jax.ShapeDtypeStruct((NC, ACC, CW), jnp.float32)],
    scratch_types=[
        pltpu.VMEM((NCH, C), jnp.int32),
        pltpu.VMEM((NCH, C), jnp.int32),
        pltpu.VMEM((C, CW), jnp.float32),
        pltpu.VMEM((RPT, CW), jnp.float32),
        pltpu.VMEM_SHARED((ACC, CW), jnp.float32),
    ],
)(_sc_counts)


def _tc_combine(x_ref, p0_ref, p1_ref, c0_ref, c1_ref, ws_ref, wn_ref, o_ref):
    s = p0_ref[0] + p1_ref[0]
    cnt = c0_ref[0, :, 0] + c1_ref[0, :, 0]
    mean = s / jnp.maximum(cnt, 1.0)[:, None]
    a = jnp.dot(x_ref[...], ws_ref[...], preferred_element_type=jnp.float32)
    b = jnp.dot(mean, wn_ref[...], preferred_element_type=jnp.float32)
    o_ref[...] = jnp.maximum(jnp.concatenate([a, b], axis=1), 0.0)


def kernel(x, edge_index, W_self, W_neigh):
    src = edge_index[0].astype(jnp.int32)
    dst = edge_index[1].astype(jnp.int32)
    e = src.shape[0]
    pad = NW * EW - e
    src = jnp.concatenate([src, jnp.zeros((pad,), jnp.int32)])
    dst = jnp.concatenate([dst, jnp.full((pad,), N_NODES, jnp.int32)])
    packed = jnp.left_shift(dst, 16) | src
    idx3 = packed.reshape(NW, NCH, C)

    (sums,) = _sums_call(x, idx3)
    (cnts,) = _cnts_call(idx3)

    return pl.pallas_call(
        _tc_combine,
        grid=(N_NODES // BLK,),
        in_specs=[
            pl.BlockSpec((BLK, D), lambda i: (i, 0)),
            pl.BlockSpec((1, BLK, D), lambda i: (0, i, 0)),
            pl.BlockSpec((1, BLK, D), lambda i: (1, i, 0)),
            pl.BlockSpec((1, BLK, CW), lambda i: (0, i, 0)),
            pl.BlockSpec((1, BLK, CW), lambda i: (1, i, 0)),
            pl.BlockSpec((D, D), lambda i: (0, 0)),
            pl.BlockSpec((D, D), lambda i: (0, 0)),
        ],
        out_specs=pl.BlockSpec((BLK, 2 * D), lambda i: (i, 0)),
        out_shape=jax.ShapeDtypeStruct((N_NODES, 2 * D), jnp.float32),
    )(x, sums, sums, cnts, cnts, W_self, W_neigh)


# speedup vs baseline: 3.5189x; 1.0361x over previous
import functools

import jax
import jax.numpy as jnp
from jax import lax
from jax.experimental import pallas as pl
from jax.experimental.pallas import tpu as pltpu
from jax.experimental.pallas import tpu_sc as plsc

N_NODES = 10000
D = 128
NC, NS = 2, 16
NW = NC * NS
EW = 10240
C = 80
NCH = EW // C
ACC = 10240
RPT = ACC // NS
CW = 16
BLK = 1000

_MESH = plsc.VectorSubcoreMesh(core_axis_name="c", subcore_axis_name="s")
_SC_PARAMS = pltpu.CompilerParams(use_tc_tiling_on_sc=False)


def _sc_feature_sums(x_hbm, idx_hbm, sums_hbm,
                     src_v, dst_v, dum_v, buf0_v, buf1_v,
                     gsem0, gsem1, ssem, acc_sh):
    cid = lax.axis_index("c")
    sid = lax.axis_index("s")
    wid = cid * NS + sid

    pltpu.sync_copy(idx_hbm.at[wid], src_v)

    def _unpack(i, _):
        r = i // (C // 16)
        c = (i % (C // 16)) * 16
        v = src_v[r, pl.ds(c, 16)]
        src_v[r, pl.ds(c, 16)] = jnp.bitwise_and(v, 0xFFFF)
        dst_v[r, pl.ds(c, 16)] = lax.shift_right_logical(v, 16)
        return 0
    lax.fori_loop(0, NCH * (C // 16), _unpack, 0)

    def _dum(i, _):
        dum_v[pl.ds(i * 16, 16)] = jnp.full((16,), N_NODES, jnp.int32)
        return 0
    lax.fori_loop(0, C // 16, _dum, 0)

    def _zrow(i, _):
        r = i // (D // 16)
        c = (i % (D // 16)) * 16
        buf0_v[r, pl.ds(c, 16)] = jnp.zeros((16,), jnp.float32)
        buf1_v[r, pl.ds(c, 16)] = jnp.zeros((16,), jnp.float32)
        return 0
    lax.fori_loop(0, C * (D // 16), _zrow, 0)

    base = sid * RPT
    for i in range(RPT // C):
        pltpu.sync_copy(buf0_v, acc_sh.at[pl.ds(base + i * C, C)])
    plsc.subcore_barrier()

    pltpu.async_copy(x_hbm.at[src_v.at[0]], buf0_v, gsem0)
    pltpu.async_copy(buf1_v, acc_sh.at[dum_v], ssem, add=True)

    def _pair(jj, _):
        c0 = jj * 2
        pltpu.make_async_copy(x_hbm.at[src_v.at[c0]], buf0_v, gsem0).wait()
        pltpu.make_async_copy(buf1_v, acc_sh.at[dum_v], ssem).wait()
        pltpu.async_copy(x_hbm.at[src_v.at[c0 + 1]], buf1_v, gsem1)
        pltpu.async_copy(buf0_v, acc_sh.at[dst_v.at[c0]], ssem, add=True)
        pltpu.make_async_copy(x_hbm.at[src_v.at[c0 + 1]], buf1_v, gsem1).wait()
        pltpu.make_async_copy(buf0_v, acc_sh.at[dum_v], ssem).wait()
        c2 = jnp.minimum(c0 + 2, NCH - 1)
        pltpu.async_copy(x_hbm.at[src_v.at[c2]], buf0_v, gsem0)
        pltpu.async_copy(buf1_v, acc_sh.at[dst_v.at[c0 + 1]], ssem, add=True)
        return 0
    lax.fori_loop(0, NCH // 2, _pair, 0)

    pltpu.make_async_copy(x_hbm.at[src_v.at[0]], buf0_v, gsem0).wait()
    pltpu.make_async_copy(buf1_v, acc_sh.at[dum_v], ssem).wait()

    plsc.subcore_barrier()

    pltpu.sync_copy(acc_sh.at[pl.ds(base, RPT)],
                    sums_hbm.at[cid, pl.ds(base, RPT)])


def _sc_counts(idx_hbm, cnts_hbm, idx_v, dst_v, ones_v, z16_v, cnt_sh):
    cid = lax.axis_index("c")
    sid = lax.axis_index("s")
    wid = cid * NS + sid

    pltpu.sync_copy(idx_hbm.at[wid], idx_v)

    def _unpack(i, _):
        r = i // (C // 16)
        c = (i % (C // 16)) * 16
        dst_v[r, pl.ds(c, 16)] = lax.shift_right_logical(
            idx_v[r, pl.ds(c, 16)], 16)
        return 0
    lax.fori_loop(0, NCH * (C // 16), _unpack, 0)

    def _fill(i, _):
        ones_v[i, :] = jnp.ones((CW,), jnp.float32)
        return 0
    lax.fori_loop(0, C, _fill, 0)

    def _z16(i, _):
        z16_v[i, :] = jnp.zeros((CW,), jnp.float32)
        return 0
    lax.fori_loop(0, RPT, _z16, 0)

    base = sid * RPT
    pltpu.sync_copy(z16_v, cnt_sh.at[pl.ds(base, RPT)])
    plsc.subcore_barrier()

    def _chunk(j, _):
        pltpu.sync_copy(ones_v, cnt_sh.at[dst_v.at[j]], add=True)
        return 0
    lax.fori_loop(0, NCH, _chunk, 0)

    plsc.subcore_barrier()

    pltpu.sync_copy(cnt_sh.at[pl.ds(base, RPT)],
                    cnts_hbm.at[cid, pl.ds(base, RPT)])


_sums_call = functools.partial(
    pl.kernel,
    mesh=_MESH,
    compiler_params=_SC_PARAMS,
    out_type=[jax.ShapeDtypeStruct((NC, ACC, D), jnp.float32)],
    scratch_types=[
        pltpu.VMEM((NCH, C), jnp.int32),
        pltpu.VMEM((NCH, C), jnp.int32),
        pltpu.VMEM((C,), jnp.int32),
        pltpu.VMEM((C, D), jnp.float32),
        pltpu.VMEM((C, D), jnp.float32),
        pltpu.SemaphoreType.DMA,
        pltpu.SemaphoreType.DMA,
        pltpu.SemaphoreType.DMA,
        pltpu.VMEM_SHARED((ACC, D), jnp.float32),
    ],
)(_sc_feature_sums)

_cnts_call = functools.partial(
    pl.kernel,
    mesh=_MESH,
    compiler_params=_SC_PARAMS,
    out_type=[jax.ShapeDtypeStruct((NC, ACC, CW), jnp.float32)],
    scratch_types=[
        pltpu.VMEM((NCH, C), jnp.int32),
        pltpu.VMEM((NCH, C), jnp.int32),
        pltpu.VMEM((C, CW), jnp.float32),
        pltpu.VMEM((RPT, CW), jnp.float32),
        pltpu.VMEM_SHARED((ACC, CW), jnp.float32),
    ],
)(_sc_counts)


def _tc_combine(x_ref, p0_ref, p1_ref, c0_ref, c1_ref, ws_ref, wn_ref, o_ref):
    s = p0_ref[0] + p1_ref[0]
    cnt = c0_ref[0, :, 0] + c1_ref[0, :, 0]
    mean = s / jnp.maximum(cnt, 1.0)[:, None]
    a = jnp.dot(x_ref[...], ws_ref[...], preferred_element_type=jnp.float32)
    b = jnp.dot(mean, wn_ref[...], preferred_element_type=jnp.float32)
    o_ref[...] = jnp.maximum(jnp.concatenate([a, b], axis=1), 0.0)


def kernel(x, edge_index, W_self, W_neigh):
    src = edge_index[0].astype(jnp.int32)
    dst = edge_index[1].astype(jnp.int32)
    e = src.shape[0]
    pad = NW * EW - e
    src = jnp.concatenate([src, jnp.zeros((pad,), jnp.int32)])
    dst = jnp.concatenate([dst, jnp.full((pad,), N_NODES, jnp.int32)])
    packed = jnp.left_shift(dst, 16) | src
    idx3 = packed.reshape(NW, NCH, C)

    (sums,) = _sums_call(x, idx3)
    (cnts,) = _cnts_call(idx3)

    return pl.pallas_call(
        _tc_combine,
        grid=(N_NODES // BLK,),
        in_specs=[
            pl.BlockSpec((BLK, D), lambda i: (i, 0)),
            pl.BlockSpec((1, BLK, D), lambda i: (0, i, 0)),
            pl.BlockSpec((1, BLK, D), lambda i: (1, i, 0)),
            pl.BlockSpec((1, BLK, CW), lambda i: (0, i, 0)),
            pl.BlockSpec((1, BLK, CW), lambda i: (1, i, 0)),
            pl.BlockSpec((D, D), lambda i: (0, 0)),
            pl.BlockSpec((D, D), lambda i: (0, 0)),
        ],
        out_specs=pl.BlockSpec((BLK, 2 * D), lambda i: (i, 0)),
        out_shape=jax.ShapeDtypeStruct((N_NODES, 2 * D), jnp.float32),
    )(x, sums, sums, cnts, cnts, W_self, W_neigh)


# speedup vs baseline: 5.4755x; 1.5560x over previous
import functools

import jax
import jax.numpy as jnp
from jax import lax
from jax.experimental import pallas as pl
from jax.experimental.pallas import tpu as pltpu
from jax.experimental.pallas import tpu_sc as plsc

N_NODES = 10000
D = 128
HD = D // 2
NC, NS = 2, 16
EW = 20480
C = 128
NCH = EW // C
NCH2 = NCH // 2
ACC = 10240
RPT = ACC // NS
CW = 16
BLK = 1000

_MESH = plsc.VectorSubcoreMesh(core_axis_name="c", subcore_axis_name="s")
_SC_PARAMS = pltpu.CompilerParams(use_tc_tiling_on_sc=False)


def _sc_aggregate(xh_hbm, idx_hbm, sums_hbm, cnts_hbm,
                  src_v, dst_v, dum_v, buf0_v, buf1_v, ones_v, z16_v,
                  gsem0, gsem1, ssem, acc_sh, cnt_sh):
    cid = lax.axis_index("c")
    sid = lax.axis_index("s")

    pltpu.sync_copy(idx_hbm.at[sid], src_v)
    srow = cid * N_NODES

    def _unpack(i, _):
        r = i // (C // 16)
        c = (i % (C // 16)) * 16
        v = src_v[r, pl.ds(c, 16)]
        src_v[r, pl.ds(c, 16)] = jnp.bitwise_and(v, 0xFFFF) + srow
        dst_v[r, pl.ds(c, 16)] = lax.shift_right_logical(v, 16)
        return 0
    lax.fori_loop(0, NCH * (C // 16), _unpack, 0)

    def _dum(i, _):
        dum_v[pl.ds(i * 16, 16)] = jnp.full((16,), N_NODES, jnp.int32)
        return 0
    lax.fori_loop(0, C // 16, _dum, 0)

    def _zrow(i, _):
        r = i // (HD // 16)
        c = (i % (HD // 16)) * 16
        buf0_v[r, pl.ds(c, 16)] = jnp.zeros((16,), jnp.float32)
        buf1_v[r, pl.ds(c, 16)] = jnp.zeros((16,), jnp.float32)
        return 0
    lax.fori_loop(0, C * (HD // 16), _zrow, 0)

    def _orow(i, _):
        ones_v[i, :] = jnp.ones((CW,), jnp.float32)
        return 0
    lax.fori_loop(0, C, _orow, 0)

    def _z16(i, _):
        z16_v[i, :] = jnp.zeros((CW,), jnp.float32)
        return 0
    lax.fori_loop(0, RPT, _z16, 0)

    base = sid * RPT
    for i in range(RPT // C):
        pltpu.sync_copy(buf0_v, acc_sh.at[pl.ds(base + i * C, C)])
    pltpu.sync_copy(z16_v, cnt_sh.at[pl.ds(base, RPT)])
    plsc.subcore_barrier()

    pltpu.async_copy(xh_hbm.at[src_v.at[0]], buf0_v, gsem0)
    pltpu.async_copy(buf1_v, acc_sh.at[dum_v], ssem, add=True)

    def _pair(jj, _):
        c0 = jj * 2
        pltpu.make_async_copy(xh_hbm.at[src_v.at[c0]], buf0_v, gsem0).wait()
        pltpu.make_async_copy(buf1_v, acc_sh.at[dum_v], ssem).wait()
        pltpu.async_copy(xh_hbm.at[src_v.at[c0 + 1]], buf1_v, gsem1)
        pltpu.async_copy(buf0_v, acc_sh.at[dst_v.at[c0]], ssem, add=True)
        pltpu.make_async_copy(xh_hbm.at[src_v.at[c0 + 1]], buf1_v, gsem1).wait()
        pltpu.make_async_copy(buf0_v, acc_sh.at[dum_v], ssem).wait()
        c2 = jnp.minimum(c0 + 2, NCH - 1)
        pltpu.async_copy(xh_hbm.at[src_v.at[c2]], buf0_v, gsem0)
        pltpu.async_copy(buf1_v, acc_sh.at[dst_v.at[c0 + 1]], ssem, add=True)
        return 0
    lax.fori_loop(0, NCH // 2, _pair, 0)

    pltpu.make_async_copy(xh_hbm.at[src_v.at[0]], buf0_v, gsem0).wait()
    pltpu.make_async_copy(buf1_v, acc_sh.at[dum_v], ssem).wait()

    cbase = cid * NCH2

    def _cnt(j, _):
        pltpu.sync_copy(ones_v, cnt_sh.at[dst_v.at[cbase + j]], add=True)
        return 0
    lax.fori_loop(0, NCH2, _cnt, 0)

    plsc.subcore_barrier()

    pltpu.sync_copy(acc_sh.at[pl.ds(base, RPT)],
                    sums_hbm.at[pl.ds(cid * ACC + base, RPT)])
    pltpu.sync_copy(cnt_sh.at[pl.ds(base, RPT)],
                    cnts_hbm.at[pl.ds(cid * ACC + base, RPT)])


_sc_call = functools.partial(
    pl.kernel,
    mesh=_MESH,
    compiler_params=_SC_PARAMS,
    out_type=[
        jax.ShapeDtypeStruct((NC * ACC, HD), jnp.float32),
        jax.ShapeDtypeStruct((NC * ACC, CW), jnp.float32),
    ],
    scratch_types=[
        pltpu.VMEM((NCH, C), jnp.int32),
        pltpu.VMEM((NCH, C), jnp.int32),
        pltpu.VMEM((C,), jnp.int32),
        pltpu.VMEM((C, HD), jnp.float32),
        pltpu.VMEM((C, HD), jnp.float32),
        pltpu.VMEM((C, CW), jnp.float32),
        pltpu.VMEM((RPT, CW), jnp.float32),
        pltpu.SemaphoreType.DMA,
        pltpu.SemaphoreType.DMA,
        pltpu.SemaphoreType.DMA,
        pltpu.VMEM_SHARED((ACC, HD), jnp.float32),
        pltpu.VMEM_SHARED((ACC, CW), jnp.float32),
    ],
)(_sc_aggregate)


def _tc_combine(x_ref, p0_ref, p1_ref, c0_ref, c1_ref, ws_ref, wn_ref, o_ref):
    s = jnp.concatenate([p0_ref[0], p1_ref[0]], axis=1)
    cnt = c0_ref[0, :, 0] + c1_ref[0, :, 0]
    mean = s / jnp.maximum(cnt, 1.0)[:, None]
    a = jnp.dot(x_ref[...], ws_ref[...], preferred_element_type=jnp.float32)
    b = jnp.dot(mean, wn_ref[...], preferred_element_type=jnp.float32)
    o_ref[...] = jnp.maximum(jnp.concatenate([a, b], axis=1), 0.0)


def kernel(x, edge_index, W_self, W_neigh):
    src = edge_index[0].astype(jnp.int32)
    dst = edge_index[1].astype(jnp.int32)
    e = src.shape[0]
    pad = NS * EW - e
    src = jnp.concatenate([src, jnp.zeros((pad,), jnp.int32)])
    dst = jnp.concatenate([dst, jnp.full((pad,), N_NODES, jnp.int32)])
    packed = jnp.left_shift(dst, 16) | src
    idx3 = packed.reshape(NS, NCH, C)

    xh = x.reshape(N_NODES, NC, HD).swapaxes(0, 1).reshape(NC * N_NODES, HD)

    sums, cnts = _sc_call(xh, idx3)
    sums = sums.reshape(NC, ACC, HD)
    cnts = cnts.reshape(NC, ACC, CW)

    return pl.pallas_call(
        _tc_combine,
        grid=(N_NODES // BLK,),
        in_specs=[
            pl.BlockSpec((BLK, D), lambda i: (i, 0)),
            pl.BlockSpec((1, BLK, HD), lambda i: (0, i, 0)),
            pl.BlockSpec((1, BLK, HD), lambda i: (1, i, 0)),
            pl.BlockSpec((1, BLK, CW), lambda i: (0, i, 0)),
            pl.BlockSpec((1, BLK, CW), lambda i: (1, i, 0)),
            pl.BlockSpec((D, D), lambda i: (0, 0)),
            pl.BlockSpec((D, D), lambda i: (0, 0)),
        ],
        out_specs=pl.BlockSpec((BLK, 2 * D), lambda i: (i, 0)),
        out_shape=jax.ShapeDtypeStruct((N_NODES, 2 * D), jnp.float32),
    )(x, sums, sums, cnts, cnts, W_self, W_neigh)
